# baseline (speedup 1.0000x reference)
import functools

import jax
import jax.numpy as jnp
from jax import lax
from jax.experimental import pallas as pl
from jax.experimental.pallas import tpu as pltpu
from jax.experimental.pallas import tpu_sc as plsc

B, S, H, P, LR = 4, 4096, 2048, 128, 8
_BLK = 1024
_NBLK = S // _BLK


def _sc_gather(base_flat, pos_flat):
    info = plsc.get_sparse_core_info()
    nc, ns = info.num_cores, info.num_subcores
    nw = nc * ns
    rows_total = B * P
    b_per_w = rows_total // nw

    mesh = plsc.VectorSubcoreMesh(core_axis_name="c", subcore_axis_name="s")

    @functools.partial(
        pl.kernel,
        out_type=jax.ShapeDtypeStruct((rows_total, H), jnp.float32),
        mesh=mesh,
        scratch_types=[
            pltpu.VMEM((b_per_w,), jnp.int32),
            pltpu.VMEM((b_per_w, H), jnp.float32),
            pltpu.SemaphoreType.DMA,
        ],
    )
    def k(base_hbm, idx_hbm, out_hbm, idx_v, rows_v, sem):
        wid = lax.axis_index("s") * nc + lax.axis_index("c")
        row0 = wid * b_per_w
        pltpu.sync_copy(idx_hbm.at[pl.ds(row0, b_per_w)], idx_v)
        batch = row0 // P
        idx_v[...] = idx_v[...] + batch * S
        pltpu.async_copy(base_hbm.at[idx_v], rows_v, sem).wait()
        pltpu.sync_copy(rows_v, out_hbm.at[pl.ds(row0, b_per_w)])

    return k(base_flat, pos_flat)


def _tc_stream(base, gathered, rotation, weights, sorted_pos, order, starts):
    def body(sp_ref, ord_ref, st_ref, base_ref, g_ref, r_ref, w_ref, out_ref,
             mix_v):
        b = pl.program_id(0)
        s = pl.program_id(1)

        @pl.when(s == 0)
        def _compute_mixed():
            g = g_ref[...]
            rot = r_ref[0, 0]
            tmp = (jnp.dot(g, w_ref[0, 0], preferred_element_type=jnp.float32)
                   - jnp.dot(g, rot, preferred_element_type=jnp.float32))
            mix_v[...] = lax.dot_general(
                tmp, rot, (((1,), (1,)), ((), ())),
                preferred_element_type=jnp.float32)

        out_ref[...] = base_ref[...]
        start = s * _BLK

        def step(j, carry):
            off = sp_ref[b, j] - start
            src = ord_ref[b, j]
            out_ref[0, pl.ds(off, 1), :] = mix_v[pl.ds(src, 1), :]
            return carry

        lax.fori_loop(st_ref[b, s], st_ref[b, s + 1], step, 0)

    return pl.pallas_call(
        body,
        grid=(B, _NBLK),
        in_specs=[
            pl.BlockSpec(memory_space=pltpu.SMEM),
            pl.BlockSpec(memory_space=pltpu.SMEM),
            pl.BlockSpec(memory_space=pltpu.SMEM),
            pl.BlockSpec((1, _BLK, H), lambda b, s: (b, s, 0)),
            pl.BlockSpec((P, H), lambda b, s: (b, 0)),
            pl.BlockSpec((1, 1, H, LR), lambda b, s: (b, 0, 0, 0)),
            pl.BlockSpec((1, 1, H, LR), lambda b, s: (b, 0, 0, 0)),
        ],
        out_specs=pl.BlockSpec((1, _BLK, H), lambda b, s: (b, s, 0)),
        out_shape=jax.ShapeDtypeStruct((B, S, H), jnp.float32),
        scratch_shapes=[
            pltpu.VMEM((P, H), jnp.float32),
        ],
    )(sorted_pos, order, starts, base, gathered, rotation, weights)


def kernel(base, intervention_positions, batch_rotation, batch_weights):
    pos = intervention_positions.astype(jnp.int32)
    gathered = _sc_gather(base.reshape(B * S, H), pos.reshape(B * P))
    key = pos // _BLK
    blocks = jnp.arange(_NBLK, dtype=jnp.int32)
    counts = (key[:, None, :] == blocks[None, :, None]).sum(-1)
    starts = jnp.concatenate(
        [jnp.zeros((B, 1), jnp.int32),
         jnp.cumsum(counts, axis=1, dtype=jnp.int32)], axis=1)
    lt = key[:, None, :] < key[:, :, None]
    same = key[:, None, :] == key[:, :, None]
    tri = jnp.tril(jnp.ones((P, P), jnp.bool_), k=-1)
    slot = (lt | (same & tri[None])).sum(-1, dtype=jnp.int32)
    onehot = slot[:, None, :] == jnp.arange(P, dtype=jnp.int32)[None, :, None]
    order = (onehot * jnp.arange(P, dtype=jnp.int32)[None, None, :]).sum(-1)
    sorted_pos = (onehot * pos[:, None, :]).sum(-1)
    return _tc_stream(base, gathered, batch_rotation, batch_weights,
                      sorted_pos, order, starts)


# speedup vs baseline: 1.0248x; 1.0248x over previous
import functools

import jax
import jax.numpy as jnp
from jax import lax
from jax.experimental import pallas as pl
from jax.experimental.pallas import tpu as pltpu
from jax.experimental.pallas import tpu_sc as plsc

B, S, H, P, LR = 4, 4096, 2048, 128, 8
_BLK = 1024
_NBLK = S // _BLK


def _sc_gather(base_flat, pos_flat):
    info = plsc.get_sparse_core_info()
    nc, ns = info.num_cores, info.num_subcores
    nw = nc * ns
    rows_total = B * P
    b_per_w = rows_total // nw

    mesh = plsc.VectorSubcoreMesh(core_axis_name="c", subcore_axis_name="s")

    @functools.partial(
        pl.kernel,
        out_type=jax.ShapeDtypeStruct((rows_total, H), jnp.float32),
        mesh=mesh,
        scratch_types=[
            pltpu.VMEM((b_per_w,), jnp.int32),
            pltpu.VMEM((b_per_w, H), jnp.float32),
            pltpu.SemaphoreType.DMA,
        ],
    )
    def k(base_hbm, idx_hbm, out_hbm, idx_v, rows_v, sem):
        wid = lax.axis_index("s") * nc + lax.axis_index("c")
        row0 = wid * b_per_w
        pltpu.sync_copy(idx_hbm.at[pl.ds(row0, b_per_w)], idx_v)
        batch = row0 // P
        idx_v[...] = idx_v[...] + batch * S
        pltpu.async_copy(base_hbm.at[idx_v], rows_v, sem).wait()
        pltpu.sync_copy(rows_v, out_hbm.at[pl.ds(row0, b_per_w)])

    return k(base_flat, pos_flat)


def _tc_stream(base, gathered, wr, sorted_pos, order, starts):
    def body(sp_ref, ord_ref, st_ref, base_ref, g_ref, wr_ref, out_ref, mix_v):
        b = pl.program_id(0)
        s = pl.program_id(1)

        @pl.when(s == 0)
        def _compute_mixed():
            g = g_ref[...]
            rot = wr_ref[0, 0, :, LR:]
            tmp = (jnp.dot(g, wr_ref[0, 0, :, :LR], preferred_element_type=jnp.float32)
                   - jnp.dot(g, rot, preferred_element_type=jnp.float32))
            mix_v[...] = lax.dot_general(
                tmp, rot, (((1,), (1,)), ((), ())),
                preferred_element_type=jnp.float32)

        out_ref[...] = base_ref[...]
        start = s * _BLK

        def step(j, carry):
            off = sp_ref[b, j] - start
            src = ord_ref[b, j]
            out_ref[0, pl.ds(off, 1), :] = mix_v[pl.ds(src, 1), :]
            return carry

        lax.fori_loop(st_ref[b, s], st_ref[b, s + 1], step, 0)

    return pl.pallas_call(
        body,
        grid=(B, _NBLK),
        in_specs=[
            pl.BlockSpec(memory_space=pltpu.SMEM),
            pl.BlockSpec(memory_space=pltpu.SMEM),
            pl.BlockSpec(memory_space=pltpu.SMEM),
            pl.BlockSpec((1, _BLK, H), lambda b, s: (b, s, 0)),
            pl.BlockSpec((P, H), lambda b, s: (b, 0)),
            pl.BlockSpec((1, 1, H, 2 * LR), lambda b, s: (b, 0, 0, 0)),
        ],
        out_specs=pl.BlockSpec((1, _BLK, H), lambda b, s: (b, s, 0)),
        out_shape=jax.ShapeDtypeStruct((B, S, H), jnp.float32),
        scratch_shapes=[
            pltpu.VMEM((P, H), jnp.float32),
        ],
    )(sorted_pos, order, starts, base, gathered, wr)


def kernel(base, intervention_positions, batch_rotation, batch_weights):
    pos = intervention_positions.astype(jnp.int32)
    gathered = _sc_gather(base.reshape(B * S, H), pos.reshape(B * P))
    key = pos // _BLK
    blocks = jnp.arange(_NBLK, dtype=jnp.int32)
    counts = (key[:, None, :] == blocks[None, :, None]).sum(-1)
    starts = jnp.concatenate(
        [jnp.zeros((B, 1), jnp.int32),
         jnp.cumsum(counts, axis=1, dtype=jnp.int32)], axis=1)
    lt = key[:, None, :] < key[:, :, None]
    same = key[:, None, :] == key[:, :, None]
    tri = jnp.tril(jnp.ones((P, P), jnp.bool_), k=-1)
    slot = (lt | (same & tri[None])).sum(-1, dtype=jnp.int32)
    onehot = slot[:, None, :] == jnp.arange(P, dtype=jnp.int32)[None, :, None]
    order = (onehot * jnp.arange(P, dtype=jnp.int32)[None, None, :]).sum(-1)
    sorted_pos = (onehot * pos[:, None, :]).sum(-1)
    wr = jnp.concatenate([batch_weights, batch_rotation], axis=-1)
    return _tc_stream(base, gathered, wr, sorted_pos, order, starts)
